# Initial kernel scaffold; baseline (speedup 1.0000x reference)
import jax
import jax.numpy as jnp
from jax.experimental import pallas as pl


def kernel(x, edge_index, W1, b1, W2, b2):
    raise NotImplementedError("write your pallas kernel here")


# speedup vs baseline: 27.7639x; 27.7639x over previous
import functools

import jax
import jax.numpy as jnp
from jax import lax
from jax.experimental import pallas as pl
from jax.experimental.pallas import tpu as pltpu
from jax.experimental.pallas import tpu_sc as plsc

_N = 10000
_NC = 2
_NS = 16
_NW = _NC * _NS
_SL = 128
_ACC_ROWS = 10240
_RPT = _ACC_ROWS // _NS

_mesh = plsc.VectorSubcoreMesh(core_axis_name="c", subcore_axis_name="s")
_sc_params = pltpu.CompilerParams(use_tc_tiling_on_sc=False)


def _deg_kernel_body(didx_hbm, zeros_hbm, ones_hbm, out_hbm,
                     didx_v, ones_v, acc_sh, sem):
    cid = lax.axis_index("c")
    sid = lax.axis_index("s")
    wid = cid * _NS + sid
    k = didx_hbm.shape[1]

    pltpu.sync_copy(zeros_hbm.at[pl.ds(0, _RPT)], acc_sh.at[pl.ds(sid * _RPT, _RPT)])
    pltpu.sync_copy(ones_hbm, ones_v)
    pltpu.sync_copy(didx_hbm.at[wid], didx_v)
    plsc.subcore_barrier()

    def body(j, carry):
        pltpu.sync_copy(ones_v, acc_sh.at[didx_v.at[j]], add=True)
        return carry

    lax.fori_loop(0, k, body, 0, unroll=False)
    plsc.subcore_barrier()
    pltpu.sync_copy(acc_sh.at[pl.ds(sid * _RPT, _RPT)],
                    out_hbm.at[cid, pl.ds(sid * _RPT, _RPT)])


def _make_deg(k_slices):
    return functools.partial(
        pl.kernel,
        out_type=jax.ShapeDtypeStruct((_NC, _ACC_ROWS), jnp.float32),
        mesh=_mesh,
        scratch_types=[
            pltpu.VMEM((k_slices, _SL), jnp.int32),
            pltpu.VMEM((_SL,), jnp.float32),
            pltpu.VMEM_SHARED((_ACC_ROWS,), jnp.float32),
            pltpu.SemaphoreType.DMA,
        ],
        compiler_params=_sc_params,
    )(_deg_kernel_body)


def _agg_kernel_body(y_hbm, sidx_hbm, didx_hbm, zeros_hbm, out_hbm,
                     sidx_v, didx_v, rows_v, acc_sh, sem):
    cid = lax.axis_index("c")
    sid = lax.axis_index("s")
    wid = cid * _NS + sid
    k = sidx_hbm.shape[1]

    pltpu.sync_copy(zeros_hbm.at[pl.ds(0, _RPT)], acc_sh.at[pl.ds(sid * _RPT, _RPT)])
    pltpu.sync_copy(sidx_hbm.at[wid], sidx_v)
    pltpu.sync_copy(didx_hbm.at[wid], didx_v)
    plsc.subcore_barrier()

    def body(j, carry):
        pltpu.async_copy(y_hbm.at[sidx_v.at[j]], rows_v, sem).wait()
        pltpu.sync_copy(rows_v, acc_sh.at[didx_v.at[j]], add=True)
        return carry

    lax.fori_loop(0, k, body, 0, unroll=False)
    plsc.subcore_barrier()
    pltpu.sync_copy(acc_sh.at[pl.ds(sid * _RPT, _RPT)],
                    out_hbm.at[cid, pl.ds(sid * _RPT, _RPT)])


def _make_agg(feat, k_slices):
    return functools.partial(
        pl.kernel,
        out_type=jax.ShapeDtypeStruct((_NC, _ACC_ROWS, feat), jnp.float32),
        mesh=_mesh,
        scratch_types=[
            pltpu.VMEM((k_slices, _SL), jnp.int32),
            pltpu.VMEM((k_slices, _SL), jnp.int32),
            pltpu.VMEM((_SL, feat), jnp.float32),
            pltpu.VMEM_SHARED((_ACC_ROWS, feat), jnp.float32),
            pltpu.SemaphoreType.DMA,
        ],
        compiler_params=_sc_params,
    )(_agg_kernel_body)


def _tc_scale1_body(x_ref, w_ref, degp_ref, y_ref, dis_ref):
    deg = degp_ref[0, : _N, :] + degp_ref[1, : _N, :] + 1.0
    dis = lax.rsqrt(deg)
    xw = jnp.dot(x_ref[...], w_ref[...], preferred_element_type=jnp.float32)
    y_ref[...] = xw * dis
    dis_ref[...] = dis


def _tc_mid_body(p_ref, y1_ref, dis_ref, b1_ref, w2_ref, y2_ref):
    dis = dis_ref[...]
    agg = p_ref[0, : _N, :] + p_ref[1, : _N, :] + y1_ref[...]
    h = jnp.maximum(agg * dis + b1_ref[...], 0.0)
    xw2 = jnp.dot(h, w2_ref[...], preferred_element_type=jnp.float32)
    y2_ref[...] = xw2 * dis


def _tc_final_body(p_ref, y2_ref, dis_ref, b2_ref, o_ref):
    agg = p_ref[0, : _N, :] + p_ref[1, : _N, :] + y2_ref[...]
    z = agg * dis_ref[...] + b2_ref[...]
    m = jnp.max(z, axis=1, keepdims=True)
    lse = jnp.log(jnp.sum(jnp.exp(z - m), axis=1, keepdims=True)) + m
    o_ref[...] = z - lse


def kernel(x, edge_index, W1, b1, W2, b2):
    n, d_in = x.shape
    d_hid = W1.shape[1]
    d_out = W2.shape[1]
    e = edge_index.shape[1]

    src = edge_index[0].astype(jnp.int32)
    dst = edge_index[1].astype(jnp.int32)
    k_slices = -(-e // (_NW * _SL))
    e_pad = _NW * _SL * k_slices
    src_p = jnp.concatenate([src, jnp.zeros((e_pad - e,), jnp.int32)])
    dst_p = jnp.concatenate([dst, jnp.full((e_pad - e,), _N, jnp.int32)])
    src3 = src_p.reshape(_NW, k_slices, _SL)
    dst3 = dst_p.reshape(_NW, k_slices, _SL)
    zeros_hbm = jnp.zeros((_RPT, max(d_hid, d_out)), jnp.float32)

    degp = _make_deg(k_slices)(dst3, zeros_hbm[:, 0], jnp.ones((_SL,), jnp.float32))

    y1, dis = pl.pallas_call(
        _tc_scale1_body,
        out_shape=(
            jax.ShapeDtypeStruct((n, d_hid), jnp.float32),
            jax.ShapeDtypeStruct((n, 1), jnp.float32),
        ),
    )(x, W1, degp.reshape(_NC, _ACC_ROWS, 1))

    p1 = _make_agg(d_hid, k_slices)(y1, src3, dst3, zeros_hbm[:, :d_hid])

    y2 = pl.pallas_call(
        _tc_mid_body,
        out_shape=jax.ShapeDtypeStruct((n, d_out), jnp.float32),
    )(p1, y1, dis, b1.reshape(1, d_hid), W2)

    p2 = _make_agg(d_out, k_slices)(y2, src3, dst3, zeros_hbm[:, :d_out])

    out = pl.pallas_call(
        _tc_final_body,
        out_shape=jax.ShapeDtypeStruct((n, d_out), jnp.float32),
    )(p2, y2, dis, b2.reshape(1, d_out))
    return out
